# baseline (speedup 1.0000x reference)
import functools
import jax
import jax.numpy as jnp
from jax import lax
from jax.experimental import pallas as pl
from jax.experimental.pallas import tpu as pltpu
from jax.experimental.pallas import tpu_sc as plsc

_NC, _NS = 2, 16
_NW = _NC * _NS


def kernel(x, cluster_labels, table):
    b, lp1, d = x.shape
    v = table.shape[0]
    zeros_col = jnp.zeros((b, 1), dtype=cluster_labels.dtype)
    labels = jnp.concatenate([zeros_col, cluster_labels], axis=1)
    labels = jnp.pad(labels, ((0, 0), (0, 55)))
    table_p = jnp.pad(table, ((0, 0), (0, 64)))

    mesh = plsc.VectorSubcoreMesh(
        core_axis_name="c", subcore_axis_name="s",
        num_cores=_NC, num_subcores=_NS)
    bpw = b // _NW

    @functools.partial(
        pl.kernel,
        out_type=jax.ShapeDtypeStruct((b, lp1, d), jnp.float32),
        mesh=mesh,
        compiler_params=pltpu.CompilerParams(use_tc_tiling_on_sc=True),
        scratch_types=(
            [pltpu.VMEM((256,), jnp.int32)]
            + [pltpu.VMEM((lp1, d), jnp.float32)]
            + [pltpu.VMEM((208, 128), jnp.float32)]
            + [pltpu.SemaphoreType.DMA]
        ),
    )
    def k(x_hbm, idx_hbm, table_hbm, out_hbm, idx_v, x_v, g_v, sem):
        wid = lax.axis_index("s") * _NC + lax.axis_index("c")

        def body(i, carry):
            cb = wid * bpw + i
            pltpu.sync_copy(idx_hbm.at[cb], idx_v)
            pltpu.sync_copy(x_hbm.at[cb], x_v)
            pltpu.async_copy(table_hbm.at[idx_v.at[pl.ds(0, 128)]],
                             g_v.at[pl.ds(0, 128)], sem).wait()
            pltpu.async_copy(table_hbm.at[idx_v.at[pl.ds(128, 73)]],
                             g_v.at[pl.ds(128, 73)], sem).wait()

            def add_row(t, c2):
                for j in range(d // 16):
                    sl = pl.ds(j * 16, 16)
                    x_v[t, sl] = x_v[t, sl] + g_v[t, sl]
                return c2

            lax.fori_loop(0, lp1, add_row, 0)
            pltpu.sync_copy(x_v, out_hbm.at[cb])
            return carry

        lax.fori_loop(0, bpw, body, 0)

    return k(x, labels, table_p)
